# Initial kernel scaffold; baseline (speedup 1.0000x reference)
import jax
import jax.numpy as jnp
from jax.experimental import pallas as pl


def kernel(p, x, ln1_g, ln1_b, Wqkv, Wproj, bproj, ln2_g, ln2_b, W1, b1, W2, b2, o):
    raise NotImplementedError("write your pallas kernel here")


# speedup vs baseline: 10.5585x; 10.5585x over previous
import jax
import jax.numpy as jnp
from jax.experimental import pallas as pl

N = 4096
C = 256
K = 16
L = 4
TILE = 256
NT = N // TILE
SCALE = float(C) ** (-0.5)
BISECT_ITERS = 20


def _knn_mask_kernel(p_ref, pt_ref, mask_ref):
    pt = p_ref[...]
    pT = pt_ref[...]
    sqt = jnp.sum(pt * pt, axis=1, keepdims=True)
    sq = jnp.sum(pT * pT, axis=0, keepdims=True)
    cross = jnp.dot(pt.astype(jnp.bfloat16), pT.astype(jnp.bfloat16),
                    preferred_element_type=jnp.float32)
    d = sqt + sq - 2.0 * cross
    hi = jnp.max(d, axis=1, keepdims=True)
    lo = jnp.full_like(hi, -1.0)

    def body(_, carry):
        lo, hi = carry
        mid = 0.5 * (lo + hi)
        cnt = jnp.sum((d <= mid).astype(jnp.float32), axis=1, keepdims=True)
        pred = cnt >= float(K)
        return jnp.where(pred, lo, mid), jnp.where(pred, mid, hi)

    lo, hi = jax.lax.fori_loop(0, BISECT_ITERS, body, (lo, hi))
    mask_ref[...] = (d <= hi).astype(jnp.bfloat16)


def _qkv_kernel(x_ref, g_ref, b_ref, w_ref, q_ref, k_ref, v_ref):
    x = x_ref[...]
    m = jnp.mean(x, axis=1, keepdims=True)
    xc = x - m
    var = jnp.mean(xc * xc, axis=1, keepdims=True)
    xn = xc * jax.lax.rsqrt(var + 1e-5) * g_ref[...] + b_ref[...]
    qkv = jnp.dot(xn.astype(jnp.bfloat16), w_ref[...],
                  preferred_element_type=jnp.float32)
    q_ref[...] = qkv[:, :C].astype(jnp.bfloat16)
    k_ref[...] = qkv[:, C:2 * C].astype(jnp.bfloat16)
    v_ref[...] = qkv[:, 2 * C:].astype(jnp.bfloat16)


def _gelu_exact(x):
    z = jnp.abs(x) * 0.7071067811865476
    t = 1.0 / (1.0 + 0.3275911 * z)
    poly = t * (0.254829592 + t * (-0.284496736 + t * (
        1.421413741 + t * (-1.453152027 + t * 1.061405429))))
    erf = 1.0 - poly * jnp.exp(-z * z)
    erf = jnp.where(x < 0.0, -erf, erf)
    return 0.5 * x * (1.0 + erf)


def _attn_mlp_kernel(x_ref, q_ref, kt_ref, v_ref, mask_ref,
                     wp_ref, bp_ref, g2_ref, b2_ref,
                     w1_ref, b1_ref, w2_ref, b2b_ref, o_ref):
    s = jnp.dot(q_ref[...], kt_ref[...],
                preferred_element_type=jnp.float32)
    u = jnp.exp(s * SCALE) * mask_ref[...].astype(jnp.float32)
    denom = jnp.sum(u, axis=1, keepdims=True)
    o = jnp.dot(u.astype(jnp.bfloat16), v_ref[...],
                preferred_element_type=jnp.float32)
    o = o / denom
    o = jnp.dot(o.astype(jnp.bfloat16), wp_ref[...],
                preferred_element_type=jnp.float32) + bp_ref[...]
    x1 = x_ref[...] + o
    m = jnp.mean(x1, axis=1, keepdims=True)
    xc = x1 - m
    var = jnp.mean(xc * xc, axis=1, keepdims=True)
    xn2 = xc * jax.lax.rsqrt(var + 1e-5) * g2_ref[...] + b2_ref[...]
    h = jnp.dot(xn2.astype(jnp.bfloat16), w1_ref[...],
                preferred_element_type=jnp.float32) + b1_ref[...]
    h = _gelu_exact(h)
    h = jnp.dot(h.astype(jnp.bfloat16), w2_ref[...],
                preferred_element_type=jnp.float32) + b2b_ref[...]
    o_ref[...] = x1 + h


def kernel(p, x, ln1_g, ln1_b, Wqkv, Wproj, bproj, ln2_g, ln2_b, W1, b1, W2,
           b2, o):
    del o
    f32 = jnp.float32
    bf16 = jnp.bfloat16
    pT = jnp.zeros((8, N), f32).at[:3, :].set(p.T)
    p8 = jnp.zeros((N, 8), f32).at[:, :3].set(p)

    mask = pl.pallas_call(
        _knn_mask_kernel,
        grid=(NT,),
        in_specs=[
            pl.BlockSpec((TILE, 8), lambda i: (i, 0)),
            pl.BlockSpec((8, N), lambda i: (0, 0)),
        ],
        out_specs=pl.BlockSpec((TILE, N), lambda i: (i, 0)),
        out_shape=jax.ShapeDtypeStruct((N, N), bf16),
    )(p8, pT)

    qkv_call = pl.pallas_call(
        _qkv_kernel,
        grid=(NT,),
        in_specs=[
            pl.BlockSpec((TILE, C), lambda i: (i, 0)),
            pl.BlockSpec((1, C), lambda i: (0, 0)),
            pl.BlockSpec((1, C), lambda i: (0, 0)),
            pl.BlockSpec((C, 3 * C), lambda i: (0, 0)),
        ],
        out_specs=[
            pl.BlockSpec((TILE, C), lambda i: (i, 0)),
            pl.BlockSpec((TILE, C), lambda i: (i, 0)),
            pl.BlockSpec((TILE, C), lambda i: (i, 0)),
        ],
        out_shape=[
            jax.ShapeDtypeStruct((N, C), bf16),
            jax.ShapeDtypeStruct((N, C), bf16),
            jax.ShapeDtypeStruct((N, C), bf16),
        ],
    )

    attn_call = pl.pallas_call(
        _attn_mlp_kernel,
        grid=(NT,),
        in_specs=[
            pl.BlockSpec((TILE, C), lambda i: (i, 0)),
            pl.BlockSpec((TILE, C), lambda i: (i, 0)),
            pl.BlockSpec((C, N), lambda i: (0, 0)),
            pl.BlockSpec((N, C), lambda i: (0, 0)),
            pl.BlockSpec((TILE, N), lambda i: (i, 0)),
            pl.BlockSpec((C, C), lambda i: (0, 0)),
            pl.BlockSpec((1, C), lambda i: (0, 0)),
            pl.BlockSpec((1, C), lambda i: (0, 0)),
            pl.BlockSpec((1, C), lambda i: (0, 0)),
            pl.BlockSpec((C, 4 * C), lambda i: (0, 0)),
            pl.BlockSpec((1, 4 * C), lambda i: (0, 0)),
            pl.BlockSpec((4 * C, C), lambda i: (0, 0)),
            pl.BlockSpec((1, C), lambda i: (0, 0)),
        ],
        out_specs=pl.BlockSpec((TILE, C), lambda i: (i, 0)),
        out_shape=jax.ShapeDtypeStruct((N, C), f32),
    )

    for i in range(L):
        q, k, v = qkv_call(x, ln1_g[i][None, :], ln1_b[i][None, :],
                           Wqkv[i].astype(bf16))
        kt = k.T
        x = attn_call(x, q, kt, v, mask,
                      Wproj[i].astype(bf16), bproj[i][None, :],
                      ln2_g[i][None, :], ln2_b[i][None, :],
                      W1[i].astype(bf16), b1[i][None, :],
                      W2[i].astype(bf16), b2[i][None, :])
    return x
